# Initial kernel scaffold; baseline (speedup 1.0000x reference)
import jax
import jax.numpy as jnp
from jax.experimental import pallas as pl


def kernel(x, edge_index, Wl0, bl0, Wr0, gamma0, beta0, Wl1, bl1, Wr1, gamma1, beta1, Wl2, bl2, Wr2, gamma2, beta2):
    raise NotImplementedError("write your pallas kernel here")


# speedup vs baseline: 2.9948x; 2.9948x over previous
import functools

import jax
import jax.numpy as jnp
from jax import lax
from jax.experimental import pallas as pl
from jax.experimental.pallas import tpu as pltpu
from jax.experimental.pallas import tpu_sc as plsc

N = 10000
D = 128
E = 320000
NC = 2
NS = 16
NW = NC * NS
N_PAD = 10240
ROWS_PER_TILE = N_PAD // NS
E_PAD = 327680
EDGES_PER_TILE = E_PAD // NW
CH = 128
CHUNKS = EDGES_PER_TILE // CH

_mesh = plsc.VectorSubcoreMesh(core_axis_name="c", subcore_axis_name="s")
_sc_params = pltpu.CompilerParams(needs_layout_passes=False)


def _sc_agg_body(compute_deg, h_hbm, src_hbm, dst_hbm, zeros_hbm, out_hbm,
                 deg_hbm, sidx, didx, rows, deg_v, acc):
    c = lax.axis_index("c")
    s = lax.axis_index("s")
    wid = c * NS + s

    pltpu.sync_copy(zeros_hbm, acc.at[pl.ds(s * ROWS_PER_TILE, ROWS_PER_TILE)])

    if compute_deg:
        @pl.loop(0, N_PAD // 16)
        def _zero_deg(i):
            deg_v[pl.ds(i * 16, 16)] = jnp.zeros((16,), jnp.float32)

    plsc.subcore_barrier()

    base = wid * EDGES_PER_TILE

    @pl.loop(0, CHUNKS)
    def _chunk(j):
        off = base + j * CH
        pltpu.sync_copy(src_hbm.at[pl.ds(off, CH)], sidx)
        pltpu.sync_copy(dst_hbm.at[pl.ds(off, CH)], didx)
        pltpu.sync_copy(h_hbm.at[sidx], rows)
        pltpu.sync_copy(rows, acc.at[didx], add=True)
        if compute_deg:
            ones16 = jnp.ones((16,), jnp.float32)

            @pl.loop(0, CH // 16)
            def _deg(k):
                idx = didx[pl.ds(k * 16, 16)]
                plsc.addupdate_scatter(deg_v, [idx], ones16)

    plsc.subcore_barrier()

    sl = pl.ds(s * ROWS_PER_TILE, ROWS_PER_TILE)
    pltpu.sync_copy(acc.at[sl], out_hbm.at[c, sl])
    if compute_deg:
        pltpu.sync_copy(deg_v, deg_hbm.at[wid])


@functools.partial(
    pl.kernel,
    mesh=_mesh,
    out_type=(
        jax.ShapeDtypeStruct((NC, N_PAD, D), jnp.float32),
        jax.ShapeDtypeStruct((NW, N_PAD), jnp.float32),
    ),
    scratch_types=[
        pltpu.VMEM((CH,), jnp.int32),
        pltpu.VMEM((CH,), jnp.int32),
        pltpu.VMEM((CH, D), jnp.float32),
        pltpu.VMEM((N_PAD,), jnp.float32),
        pltpu.VMEM_SHARED((N_PAD, D), jnp.float32),
    ],
    compiler_params=_sc_params,
)
def _sc_agg_deg(h_hbm, src_hbm, dst_hbm, zeros_hbm, out_hbm, deg_hbm,
                sidx, didx, rows, deg_v, acc):
    _sc_agg_body(True, h_hbm, src_hbm, dst_hbm, zeros_hbm, out_hbm,
                 deg_hbm, sidx, didx, rows, deg_v, acc)


@functools.partial(
    pl.kernel,
    mesh=_mesh,
    out_type=jax.ShapeDtypeStruct((NC, N_PAD, D), jnp.float32),
    scratch_types=[
        pltpu.VMEM((CH,), jnp.int32),
        pltpu.VMEM((CH,), jnp.int32),
        pltpu.VMEM((CH, D), jnp.float32),
        pltpu.VMEM_SHARED((N_PAD, D), jnp.float32),
    ],
    compiler_params=_sc_params,
)
def _sc_agg(h_hbm, src_hbm, dst_hbm, zeros_hbm, out_hbm,
            sidx, didx, rows, acc):
    _sc_agg_body(False, h_hbm, src_hbm, dst_hbm, zeros_hbm, out_hbm,
                 None, sidx, didx, rows, None, acc)


def _dense_body(p0, p1, degT, h, Wl, bl, Wr, gamma, beta, out):
    deg = jnp.sum(degT[...], axis=1, keepdims=True)
    agg = (p0[...] + p1[...]) / jnp.maximum(deg, 1.0)
    y = (jnp.dot(agg, Wl[...], preferred_element_type=jnp.float32)
         + jnp.dot(h[...], Wr[...], preferred_element_type=jnp.float32)
         + bl[...][None, :])
    mean = jnp.mean(y, axis=0, keepdims=True)
    var = jnp.mean((y - mean) ** 2, axis=0, keepdims=True)
    yn = (y - mean) * lax.rsqrt(var + 1e-5) * gamma[...][None, :] + beta[...][None, :]
    out[...] = jnp.maximum(yn, 0.0)


def _dense(p0, p1, degT, h, Wl, bl, Wr, gamma, beta):
    return pl.pallas_call(
        _dense_body,
        out_shape=jax.ShapeDtypeStruct((N, D), jnp.float32),
    )(p0, p1, degT, h, Wl, bl, Wr, gamma, beta)


def kernel(x, edge_index, Wl0, bl0, Wr0, gamma0, beta0, Wl1, bl1, Wr1,
           gamma1, beta1, Wl2, bl2, Wr2, gamma2, beta2):
    src = edge_index[0]
    dst = edge_index[1]
    pad = E_PAD - E
    src_p = jnp.concatenate([src, jnp.zeros((pad,), jnp.int32)])
    dst_p = jnp.concatenate([dst, jnp.full((pad,), N_PAD - 1, jnp.int32)])
    zeros = jnp.zeros((ROWS_PER_TILE, D), jnp.float32)

    params = [
        (Wl0, bl0, Wr0, gamma0, beta0),
        (Wl1, bl1, Wr1, gamma1, beta1),
        (Wl2, bl2, Wr2, gamma2, beta2),
    ]

    h = x
    degT = None
    for i, (Wl, bl, Wr, gamma, beta) in enumerate(params):
        if i == 0:
            parts, deg32 = _sc_agg_deg(h, src_p, dst_p, zeros)
            degT = deg32[:, :N].T
        else:
            parts = _sc_agg(h, src_p, dst_p, zeros)
        h = _dense(parts[0, :N], parts[1, :N], degT, h, Wl, bl, Wr,
                   gamma, beta)
    return h


# speedup vs baseline: 3.0054x; 1.0035x over previous
import functools

import jax
import jax.numpy as jnp
from jax import lax
from jax.experimental import pallas as pl
from jax.experimental.pallas import tpu as pltpu
from jax.experimental.pallas import tpu_sc as plsc

N = 10000
D = 128
E = 320000
NC = 2
NS = 16
NW = NC * NS
N_PAD = 10240
ROWS_PER_TILE = N_PAD // NS
E_PAD = 327680
EDGES_PER_TILE = E_PAD // NW
CH = 128
CHUNKS = EDGES_PER_TILE // CH

_mesh = plsc.VectorSubcoreMesh(core_axis_name="c", subcore_axis_name="s")
_sc_params = pltpu.CompilerParams(needs_layout_passes=False)


def _sc_agg_body(compute_deg, h_hbm, src_hbm, dst_hbm, zeros_hbm, out_hbm,
                 deg_hbm, sidx, didx, rows, deg_v, acc):
    c = lax.axis_index("c")
    s = lax.axis_index("s")
    wid = c * NS + s

    pltpu.sync_copy(zeros_hbm, acc.at[pl.ds(s * ROWS_PER_TILE, ROWS_PER_TILE)])

    if compute_deg:
        @pl.loop(0, N_PAD // 16)
        def _zero_deg(i):
            deg_v[pl.ds(i * 16, 16)] = jnp.zeros((16,), jnp.float32)

    plsc.subcore_barrier()

    base = wid * EDGES_PER_TILE

    @pl.loop(0, CHUNKS)
    def _chunk(j):
        off = base + j * CH
        pltpu.sync_copy(src_hbm.at[pl.ds(off, CH)], sidx)
        pltpu.sync_copy(dst_hbm.at[pl.ds(off, CH)], didx)
        pltpu.sync_copy(h_hbm.at[sidx], rows)
        pltpu.sync_copy(rows, acc.at[didx], add=True)
        if compute_deg:
            ones16 = jnp.ones((16,), jnp.float32)

            @pl.loop(0, CH // 16)
            def _deg(k):
                idx = didx[pl.ds(k * 16, 16)]
                plsc.addupdate_scatter(deg_v, [idx], ones16)

    plsc.subcore_barrier()

    sl = pl.ds(s * ROWS_PER_TILE, ROWS_PER_TILE)
    pltpu.sync_copy(acc.at[sl], out_hbm.at[c, sl])
    if compute_deg:
        pltpu.sync_copy(deg_v, deg_hbm.at[wid])


@functools.partial(
    pl.kernel,
    mesh=_mesh,
    out_type=(
        jax.ShapeDtypeStruct((NC, N_PAD, D), jnp.float32),
        jax.ShapeDtypeStruct((NW, N_PAD), jnp.float32),
    ),
    scratch_types=[
        pltpu.VMEM((CH,), jnp.int32),
        pltpu.VMEM((CH,), jnp.int32),
        pltpu.VMEM((CH, D), jnp.float32),
        pltpu.VMEM((N_PAD,), jnp.float32),
        pltpu.VMEM_SHARED((N_PAD, D), jnp.float32),
    ],
    compiler_params=_sc_params,
)
def _sc_agg_deg(h_hbm, src_hbm, dst_hbm, zeros_hbm, out_hbm, deg_hbm,
                sidx, didx, rows, deg_v, acc):
    _sc_agg_body(True, h_hbm, src_hbm, dst_hbm, zeros_hbm, out_hbm,
                 deg_hbm, sidx, didx, rows, deg_v, acc)


@functools.partial(
    pl.kernel,
    mesh=_mesh,
    out_type=jax.ShapeDtypeStruct((NC, N_PAD, D), jnp.float32),
    scratch_types=[
        pltpu.VMEM((CH,), jnp.int32),
        pltpu.VMEM((CH,), jnp.int32),
        pltpu.VMEM((CH, D), jnp.float32),
        pltpu.VMEM_SHARED((N_PAD, D), jnp.float32),
    ],
    compiler_params=_sc_params,
)
def _sc_agg(h_hbm, src_hbm, dst_hbm, zeros_hbm, out_hbm,
            sidx, didx, rows, acc):
    _sc_agg_body(False, h_hbm, src_hbm, dst_hbm, zeros_hbm, out_hbm,
                 None, sidx, didx, rows, None, acc)


def _dense_body(p0, p1, degT, h, Wl, bl, Wr, gamma, beta, out):
    deg = jnp.sum(degT[...], axis=1, keepdims=True)
    agg = (p0[...] + p1[...]) / jnp.maximum(deg, 1.0)
    y = (jnp.dot(agg, Wl[...], preferred_element_type=jnp.float32)
         + jnp.dot(h[...], Wr[...], preferred_element_type=jnp.float32)
         + bl[...][None, :])
    mean = jnp.mean(y, axis=0, keepdims=True)
    var = jnp.mean((y - mean) ** 2, axis=0, keepdims=True)
    yn = (y - mean) * lax.rsqrt(var + 1e-5) * gamma[...][None, :] + beta[...][None, :]
    out[...] = jnp.maximum(yn, 0.0)


def _dense(p0, p1, degT, h, Wl, bl, Wr, gamma, beta):
    return pl.pallas_call(
        _dense_body,
        out_shape=jax.ShapeDtypeStruct((N, D), jnp.float32),
    )(p0, p1, degT, h, Wl, bl, Wr, gamma, beta)


def kernel(x, edge_index, Wl0, bl0, Wr0, gamma0, beta0, Wl1, bl1, Wr1,
           gamma1, beta1, Wl2, bl2, Wr2, gamma2, beta2):
    src = edge_index[0]
    dst = edge_index[1]
    pad = E_PAD - E
    src_p = jnp.concatenate([src, jnp.zeros((pad,), jnp.int32)])
    pad_dst = N + jnp.arange(pad, dtype=jnp.int32) % (N_PAD - N)
    dst_p = jnp.concatenate([dst, pad_dst])
    zeros = jnp.zeros((ROWS_PER_TILE, D), jnp.float32)

    params = [
        (Wl0, bl0, Wr0, gamma0, beta0),
        (Wl1, bl1, Wr1, gamma1, beta1),
        (Wl2, bl2, Wr2, gamma2, beta2),
    ]

    h = x
    degT = None
    for i, (Wl, bl, Wr, gamma, beta) in enumerate(params):
        if i == 0:
            parts, deg32 = _sc_agg_deg(h, src_p, dst_p, zeros)
            degT = deg32[:, :N].T
        else:
            parts = _sc_agg(h, src_p, dst_p, zeros)
        h = _dense(parts[0, :N], parts[1, :N], degT, h, Wl, bl, Wr,
                   gamma, beta)
    return h


# speedup vs baseline: 3.8020x; 1.2651x over previous
import functools

import jax
import jax.numpy as jnp
from jax import lax
from jax.experimental import pallas as pl
from jax.experimental.pallas import tpu as pltpu
from jax.experimental.pallas import tpu_sc as plsc

N = 10000
D = 128
E = 320000
NC = 2
NS = 16
NW = NC * NS
N_PAD = 10240
ROWS_PER_TILE = N_PAD // NS
E_PAD = 327680
EDGES_PER_TILE = E_PAD // NW
CH = 128
CHUNKS = EDGES_PER_TILE // CH

_mesh = plsc.VectorSubcoreMesh(core_axis_name="c", subcore_axis_name="s")
_sc_params = pltpu.CompilerParams(needs_layout_passes=False)


NROW = 2
NIDX = 4


def _sc_agg_body(compute_deg, h_hbm, srcI_hbm, dstI_hbm, zeros_hbm, out_hbm,
                 deg_hbm, sidx, didx, rows, deg_v, acc, isem, rsem):
    c = lax.axis_index("c")
    s = lax.axis_index("s")
    wid = c * NS + s

    pltpu.sync_copy(zeros_hbm, acc.at[pl.ds(s * ROWS_PER_TILE, ROWS_PER_TILE)])

    if compute_deg:
        @pl.loop(0, N_PAD // 16)
        def _zero_deg(i):
            deg_v[pl.ds(i * 16, 16)] = jnp.zeros((16,), jnp.float32)

    def issue_idx(jj, slot):
        pltpu.async_copy(srcI_hbm.at[wid, jj], sidx.at[slot], isem.at[slot])
        pltpu.async_copy(dstI_hbm.at[wid, jj], didx.at[slot], isem.at[slot])

    def wait_idx(jj, slot):
        pltpu.make_async_copy(srcI_hbm.at[wid, jj], sidx.at[slot],
                              isem.at[slot]).wait()
        pltpu.make_async_copy(dstI_hbm.at[wid, jj], didx.at[slot],
                              isem.at[slot]).wait()

    def issue_gather(slot_i, slot_r):
        pltpu.async_copy(h_hbm.at[sidx.at[slot_i]], rows.at[slot_r],
                         rsem.at[slot_r])

    def wait_gather(slot_i, slot_r):
        pltpu.make_async_copy(h_hbm.at[sidx.at[slot_i]], rows.at[slot_r],
                              rsem.at[slot_r]).wait()

    plsc.subcore_barrier()

    for j in range(NIDX):
        issue_idx(j, j)
    for j in range(NROW):
        wait_idx(j, j)
        issue_gather(j, j)

    if compute_deg:
        ones16 = jnp.ones((16,), jnp.float32)

    @pl.loop(0, CHUNKS, step=NIDX)
    def _group(j0):
        for b4 in range(NIDX):
            j = j0 + b4
            br = b4 % NROW
            wait_gather(b4, br)
            pltpu.sync_copy(rows.at[br], acc.at[didx.at[b4]], add=True)
            if compute_deg:
                @pl.loop(0, CH // 16)
                def _deg(k):
                    idx = didx[b4, pl.ds(k * 16, 16)]
                    plsc.addupdate_scatter(deg_v, [idx], ones16)

            @pl.when(j + NROW < CHUNKS)
            def _():
                wait_idx(j + NROW, (b4 + NROW) % NIDX)
                issue_gather((b4 + NROW) % NIDX, br)

            @pl.when(j + NIDX < CHUNKS)
            def _():
                issue_idx(j + NIDX, b4)

    plsc.subcore_barrier()

    sl = pl.ds(s * ROWS_PER_TILE, ROWS_PER_TILE)
    pltpu.sync_copy(acc.at[sl], out_hbm.at[c, sl])
    if compute_deg:
        pltpu.sync_copy(deg_v, deg_hbm.at[wid])


@functools.partial(
    pl.kernel,
    mesh=_mesh,
    out_type=(
        jax.ShapeDtypeStruct((NC, N_PAD, D), jnp.float32),
        jax.ShapeDtypeStruct((NW, N_PAD), jnp.float32),
    ),
    scratch_types=[
        pltpu.VMEM((NIDX, CH), jnp.int32),
        pltpu.VMEM((NIDX, CH), jnp.int32),
        pltpu.VMEM((NROW, CH, D), jnp.float32),
        pltpu.VMEM((N_PAD,), jnp.float32),
        pltpu.VMEM_SHARED((N_PAD, D), jnp.float32),
        pltpu.SemaphoreType.DMA((NIDX,)),
        pltpu.SemaphoreType.DMA((NROW,)),
    ],
    compiler_params=_sc_params,
)
def _sc_agg_deg(h_hbm, srcI_hbm, dstI_hbm, zeros_hbm, out_hbm, deg_hbm,
                sidx, didx, rows, deg_v, acc, isem, rsem):
    _sc_agg_body(True, h_hbm, srcI_hbm, dstI_hbm, zeros_hbm, out_hbm,
                 deg_hbm, sidx, didx, rows, deg_v, acc, isem, rsem)


@functools.partial(
    pl.kernel,
    mesh=_mesh,
    out_type=jax.ShapeDtypeStruct((NC, N_PAD, D), jnp.float32),
    scratch_types=[
        pltpu.VMEM((NIDX, CH), jnp.int32),
        pltpu.VMEM((NIDX, CH), jnp.int32),
        pltpu.VMEM((NROW, CH, D), jnp.float32),
        pltpu.VMEM_SHARED((N_PAD, D), jnp.float32),
        pltpu.SemaphoreType.DMA((NIDX,)),
        pltpu.SemaphoreType.DMA((NROW,)),
    ],
    compiler_params=_sc_params,
)
def _sc_agg(h_hbm, srcI_hbm, dstI_hbm, zeros_hbm, out_hbm,
            sidx, didx, rows, acc, isem, rsem):
    _sc_agg_body(False, h_hbm, srcI_hbm, dstI_hbm, zeros_hbm, out_hbm,
                 None, sidx, didx, rows, None, acc, isem, rsem)


def _dense_body(p0, p1, degT, h, Wl, bl, Wr, gamma, beta, out):
    deg = jnp.sum(degT[...], axis=1, keepdims=True)
    agg = (p0[...] + p1[...]) / jnp.maximum(deg, 1.0)
    y = (jnp.dot(agg, Wl[...], preferred_element_type=jnp.float32)
         + jnp.dot(h[...], Wr[...], preferred_element_type=jnp.float32)
         + bl[...][None, :])
    mean = jnp.mean(y, axis=0, keepdims=True)
    var = jnp.mean((y - mean) ** 2, axis=0, keepdims=True)
    yn = (y - mean) * lax.rsqrt(var + 1e-5) * gamma[...][None, :] + beta[...][None, :]
    out[...] = jnp.maximum(yn, 0.0)


def _dense(p0, p1, degT, h, Wl, bl, Wr, gamma, beta):
    return pl.pallas_call(
        _dense_body,
        out_shape=jax.ShapeDtypeStruct((N, D), jnp.float32),
    )(p0, p1, degT, h, Wl, bl, Wr, gamma, beta)


def kernel(x, edge_index, Wl0, bl0, Wr0, gamma0, beta0, Wl1, bl1, Wr1,
           gamma1, beta1, Wl2, bl2, Wr2, gamma2, beta2):
    src = edge_index[0]
    dst = edge_index[1]
    pad = E_PAD - E
    src_p = jnp.concatenate([src, jnp.zeros((pad,), jnp.int32)])
    pad_dst = N + jnp.arange(pad, dtype=jnp.int32) % (N_PAD - N)
    dst_p = jnp.concatenate([dst, pad_dst])
    srcI = src_p.reshape(NW, CHUNKS, CH)
    dstI = dst_p.reshape(NW, CHUNKS, CH)
    zeros = jnp.zeros((ROWS_PER_TILE, D), jnp.float32)

    params = [
        (Wl0, bl0, Wr0, gamma0, beta0),
        (Wl1, bl1, Wr1, gamma1, beta1),
        (Wl2, bl2, Wr2, gamma2, beta2),
    ]

    h = x
    degT = None
    for i, (Wl, bl, Wr, gamma, beta) in enumerate(params):
        if i == 0:
            parts, deg32 = _sc_agg_deg(h, srcI, dstI, zeros)
            degT = deg32[:, :N].T
        else:
            parts = _sc_agg(h, srcI, dstI, zeros)
        h = _dense(parts[0, :N], parts[1, :N], degT, h, Wl, bl, Wr,
                   gamma, beta)
    return h


# speedup vs baseline: 4.1078x; 1.0804x over previous
import functools

import jax
import jax.numpy as jnp
from jax import lax
from jax.experimental import pallas as pl
from jax.experimental.pallas import tpu as pltpu
from jax.experimental.pallas import tpu_sc as plsc

N = 10000
D = 128
E = 320000
NC = 2
NS = 16
NW = NC * NS
N_PAD = 10240
ROWS_PER_TILE = N_PAD // NS
CH = 128
T0 = 124
T1 = 36
E0 = NS * T0 * CH
E_PAD = E0 + NS * T1 * CH

_mesh = plsc.VectorSubcoreMesh(core_axis_name="c", subcore_axis_name="s")
_sc_params = pltpu.CompilerParams(needs_layout_passes=False)


NROW = 2
NIDX = 4


def _sc_agg_body(compute_deg, h_hbm, src_hbm, dst_hbm, zeros_hbm, out_hbm,
                 deg_hbm, sidx, didx, rows, deg_v, acc, isem, rsem):
    c = lax.axis_index("c")
    s = lax.axis_index("s")
    wid = c * NS + s

    nchunks = jnp.where(c == 0, T0, T1)
    chunk0 = jnp.where(c == 0, s * T0, NS * T0 + s * T1)

    pltpu.sync_copy(zeros_hbm, acc.at[pl.ds(s * ROWS_PER_TILE, ROWS_PER_TILE)])

    if compute_deg:
        @pl.loop(0, N_PAD // 16)
        def _zero_deg(i):
            deg_v[pl.ds(i * 16, 16)] = jnp.zeros((16,), jnp.float32)

    def issue_idx(jj, slot):
        off = (chunk0 + jj) * CH
        pltpu.async_copy(src_hbm.at[pl.ds(off, CH)], sidx.at[slot],
                         isem.at[slot])
        pltpu.async_copy(dst_hbm.at[pl.ds(off, CH)], didx.at[slot],
                         isem.at[slot])

    def wait_idx(jj, slot):
        off = (chunk0 + jj) * CH
        pltpu.make_async_copy(src_hbm.at[pl.ds(off, CH)], sidx.at[slot],
                              isem.at[slot]).wait()
        pltpu.make_async_copy(dst_hbm.at[pl.ds(off, CH)], didx.at[slot],
                              isem.at[slot]).wait()

    def issue_gather(slot_i, slot_r):
        pltpu.async_copy(h_hbm.at[sidx.at[slot_i]], rows.at[slot_r],
                         rsem.at[slot_r])

    def wait_gather(slot_i, slot_r):
        pltpu.make_async_copy(h_hbm.at[sidx.at[slot_i]], rows.at[slot_r],
                              rsem.at[slot_r]).wait()

    plsc.subcore_barrier()

    for j in range(NIDX):
        issue_idx(j, j)
    for j in range(NROW):
        wait_idx(j, j)
        issue_gather(j, j)

    if compute_deg:
        ones16 = jnp.ones((16,), jnp.float32)

    @pl.loop(0, nchunks, step=NIDX)
    def _group(j0):
        for b4 in range(NIDX):
            j = j0 + b4
            br = b4 % NROW
            wait_gather(b4, br)
            pltpu.sync_copy(rows.at[br], acc.at[didx.at[b4]], add=True)
            if compute_deg:
                @pl.loop(0, CH // 16)
                def _deg(k):
                    idx = didx[b4, pl.ds(k * 16, 16)]
                    plsc.addupdate_scatter(deg_v, [idx], ones16)

            @pl.when(j + NROW < nchunks)
            def _():
                wait_idx(j + NROW, (b4 + NROW) % NIDX)
                issue_gather((b4 + NROW) % NIDX, br)

            @pl.when(j + NIDX < nchunks)
            def _():
                issue_idx(j + NIDX, b4)

    plsc.subcore_barrier()

    sl = pl.ds(s * ROWS_PER_TILE, ROWS_PER_TILE)
    pltpu.sync_copy(acc.at[sl], out_hbm.at[c, sl])
    if compute_deg:
        pltpu.sync_copy(deg_v, deg_hbm.at[wid])


@functools.partial(
    pl.kernel,
    mesh=_mesh,
    out_type=(
        jax.ShapeDtypeStruct((NC, N_PAD, D), jnp.float32),
        jax.ShapeDtypeStruct((NW, N_PAD), jnp.float32),
    ),
    scratch_types=[
        pltpu.VMEM((NIDX, CH), jnp.int32),
        pltpu.VMEM((NIDX, CH), jnp.int32),
        pltpu.VMEM((NROW, CH, D), jnp.float32),
        pltpu.VMEM((N_PAD,), jnp.float32),
        pltpu.VMEM_SHARED((N_PAD, D), jnp.float32),
        pltpu.SemaphoreType.DMA((NIDX,)),
        pltpu.SemaphoreType.DMA((NROW,)),
    ],
    compiler_params=_sc_params,
)
def _sc_agg_deg(h_hbm, srcI_hbm, dstI_hbm, zeros_hbm, out_hbm, deg_hbm,
                sidx, didx, rows, deg_v, acc, isem, rsem):
    _sc_agg_body(True, h_hbm, srcI_hbm, dstI_hbm, zeros_hbm, out_hbm,
                 deg_hbm, sidx, didx, rows, deg_v, acc, isem, rsem)


@functools.partial(
    pl.kernel,
    mesh=_mesh,
    out_type=jax.ShapeDtypeStruct((NC, N_PAD, D), jnp.float32),
    scratch_types=[
        pltpu.VMEM((NIDX, CH), jnp.int32),
        pltpu.VMEM((NIDX, CH), jnp.int32),
        pltpu.VMEM((NROW, CH, D), jnp.float32),
        pltpu.VMEM_SHARED((N_PAD, D), jnp.float32),
        pltpu.SemaphoreType.DMA((NIDX,)),
        pltpu.SemaphoreType.DMA((NROW,)),
    ],
    compiler_params=_sc_params,
)
def _sc_agg(h_hbm, srcI_hbm, dstI_hbm, zeros_hbm, out_hbm,
            sidx, didx, rows, acc, isem, rsem):
    _sc_agg_body(False, h_hbm, srcI_hbm, dstI_hbm, zeros_hbm, out_hbm,
                 None, sidx, didx, rows, None, acc, isem, rsem)


def _dense_body(parts, degT, h, Wl, bl, Wr, gamma, beta, out):
    deg = jnp.sum(degT[...], axis=1, keepdims=True)
    agg = (parts[0, :N, :] + parts[1, :N, :]) / jnp.maximum(deg, 1.0)
    y = (jnp.dot(agg, Wl[...], preferred_element_type=jnp.float32)
         + jnp.dot(h[...], Wr[...], preferred_element_type=jnp.float32)
         + bl[...][None, :])
    mean = jnp.mean(y, axis=0, keepdims=True)
    var = jnp.mean((y - mean) ** 2, axis=0, keepdims=True)
    yn = (y - mean) * lax.rsqrt(var + 1e-5) * gamma[...][None, :] + beta[...][None, :]
    out[...] = jnp.maximum(yn, 0.0)


def _dense(parts, degT, h, Wl, bl, Wr, gamma, beta):
    return pl.pallas_call(
        _dense_body,
        out_shape=jax.ShapeDtypeStruct((N, D), jnp.float32),
    )(parts, degT, h, Wl, bl, Wr, gamma, beta)


def kernel(x, edge_index, Wl0, bl0, Wr0, gamma0, beta0, Wl1, bl1, Wr1,
           gamma1, beta1, Wl2, bl2, Wr2, gamma2, beta2):
    src = edge_index[0]
    dst = edge_index[1]
    pad = E_PAD - E
    src_p = jnp.concatenate([src, jnp.zeros((pad,), jnp.int32)])
    pad_dst = N + jnp.arange(pad, dtype=jnp.int32) % (N_PAD - N)
    dst_p = jnp.concatenate([dst, pad_dst])
    zeros = jnp.zeros((ROWS_PER_TILE, D), jnp.float32)

    params = [
        (Wl0, bl0, Wr0, gamma0, beta0),
        (Wl1, bl1, Wr1, gamma1, beta1),
        (Wl2, bl2, Wr2, gamma2, beta2),
    ]

    h = x
    degT = None
    for i, (Wl, bl, Wr, gamma, beta) in enumerate(params):
        if i == 0:
            parts, deg32 = _sc_agg_deg(h, src_p, dst_p, zeros)
            degT = deg32[:, :N].T
        else:
            parts = _sc_agg(h, src_p, dst_p, zeros)
        h = _dense(parts, degT, h, Wl, bl, Wr, gamma, beta)
    return h


# speedup vs baseline: 8.8958x; 2.1656x over previous
import functools

import jax
import jax.numpy as jnp
from jax import lax
from jax.experimental import pallas as pl
from jax.experimental.pallas import tpu as pltpu
from jax.experimental.pallas import tpu_sc as plsc

N = 10000
D = 128
E = 320000
NC = 2
NS = 16
NW = NC * NS
N_PAD = 10240
ROWS_PER_TILE = N_PAD // NS
CH = 128
T0 = 144
E0 = NS * T0 * CH
C1 = (E - E0) // CH
T1_HI = C1 - 12 * NS
assert E0 + C1 * CH == E and 0 <= T1_HI <= NS

_mesh = plsc.VectorSubcoreMesh(core_axis_name="c", subcore_axis_name="s")
_sc_params = pltpu.CompilerParams(needs_layout_passes=False)


NROW = 2
NIDX = 4


def _sc_agg_body(compute_deg, h_hbm, src_hbm, dst_hbm, out_hbm,
                 deg_hbm, sidx, didx, rows, deg_v, acc, isem, rsem):
    c = lax.axis_index("c")
    s = lax.axis_index("s")
    wid = c * NS + s

    nchunks = jnp.where(c == 0, T0, 12 + (s < T1_HI))
    chunk0 = jnp.where(c == 0, s * T0,
                       NS * T0 + 12 * s + jnp.minimum(s, T1_HI))

    z16 = jnp.zeros((16,), jnp.float32)

    @pl.loop(0, CH)
    def _zrow(i):
        for k in range(D // 16):
            rows[0, i, pl.ds(k * 16, 16)] = z16

    for k in range(ROWS_PER_TILE // CH):
        pltpu.sync_copy(rows.at[0],
                        acc.at[pl.ds(s * ROWS_PER_TILE + k * CH, CH)])

    if compute_deg:
        @pl.loop(0, N_PAD // 16)
        def _zero_deg(i):
            deg_v[pl.ds(i * 16, 16)] = jnp.zeros((16,), jnp.float32)

    def issue_idx(jj, slot):
        off = (chunk0 + jj) * CH
        pltpu.async_copy(src_hbm.at[pl.ds(off, CH)], sidx.at[slot],
                         isem.at[slot])
        pltpu.async_copy(dst_hbm.at[pl.ds(off, CH)], didx.at[slot],
                         isem.at[slot])

    def wait_idx(jj, slot):
        off = (chunk0 + jj) * CH
        pltpu.make_async_copy(src_hbm.at[pl.ds(off, CH)], sidx.at[slot],
                              isem.at[slot]).wait()
        pltpu.make_async_copy(dst_hbm.at[pl.ds(off, CH)], didx.at[slot],
                              isem.at[slot]).wait()

    def issue_gather(slot_i, slot_r):
        pltpu.async_copy(h_hbm.at[sidx.at[slot_i]], rows.at[slot_r],
                         rsem.at[slot_r])

    def wait_gather(slot_i, slot_r):
        pltpu.make_async_copy(h_hbm.at[sidx.at[slot_i]], rows.at[slot_r],
                              rsem.at[slot_r]).wait()

    plsc.subcore_barrier()

    for j in range(NIDX):
        issue_idx(j, j)
    for j in range(NROW):
        wait_idx(j, j)
        issue_gather(j, j)

    if compute_deg:
        ones16 = jnp.ones((16,), jnp.float32)

    @pl.loop(0, nchunks, step=NIDX)
    def _group(j0):
        for b4 in range(NIDX):
            j = j0 + b4

            @pl.when(j < nchunks)
            def _():
                br = b4 % NROW
                wait_gather(b4, br)
                pltpu.sync_copy(rows.at[br], acc.at[didx.at[b4]], add=True)
                if compute_deg:
                    @pl.loop(0, CH // 16)
                    def _deg(k):
                        idx = didx[b4, pl.ds(k * 16, 16)]
                        plsc.addupdate_scatter(deg_v, [idx], ones16)

                @pl.when(j + NROW < nchunks)
                def _():
                    wait_idx(j + NROW, (b4 + NROW) % NIDX)
                    issue_gather((b4 + NROW) % NIDX, br)

                @pl.when(j + NIDX < nchunks)
                def _():
                    issue_idx(j + NIDX, b4)

    plsc.subcore_barrier()

    sl = pl.ds(s * ROWS_PER_TILE, ROWS_PER_TILE)
    pltpu.sync_copy(acc.at[sl], out_hbm.at[c, sl])
    if compute_deg:
        pltpu.sync_copy(deg_v, deg_hbm.at[wid])


@functools.partial(
    pl.kernel,
    mesh=_mesh,
    out_type=(
        jax.ShapeDtypeStruct((NC, N_PAD, D), jnp.float32),
        jax.ShapeDtypeStruct((NW, N_PAD), jnp.float32),
    ),
    scratch_types=[
        pltpu.VMEM((NIDX, CH), jnp.int32),
        pltpu.VMEM((NIDX, CH), jnp.int32),
        pltpu.VMEM((NROW, CH, D), jnp.float32),
        pltpu.VMEM((N_PAD,), jnp.float32),
        pltpu.VMEM_SHARED((N_PAD, D), jnp.float32),
        pltpu.SemaphoreType.DMA((NIDX,)),
        pltpu.SemaphoreType.DMA((NROW,)),
    ],
    compiler_params=_sc_params,
)
def _sc_agg_deg(h_hbm, src_hbm, dst_hbm, out_hbm, deg_hbm,
                sidx, didx, rows, deg_v, acc, isem, rsem):
    _sc_agg_body(True, h_hbm, src_hbm, dst_hbm, out_hbm,
                 deg_hbm, sidx, didx, rows, deg_v, acc, isem, rsem)


@functools.partial(
    pl.kernel,
    mesh=_mesh,
    out_type=jax.ShapeDtypeStruct((NC, N_PAD, D), jnp.float32),
    scratch_types=[
        pltpu.VMEM((NIDX, CH), jnp.int32),
        pltpu.VMEM((NIDX, CH), jnp.int32),
        pltpu.VMEM((NROW, CH, D), jnp.float32),
        pltpu.VMEM_SHARED((N_PAD, D), jnp.float32),
        pltpu.SemaphoreType.DMA((NIDX,)),
        pltpu.SemaphoreType.DMA((NROW,)),
    ],
    compiler_params=_sc_params,
)
def _sc_agg(h_hbm, src_hbm, dst_hbm, out_hbm,
            sidx, didx, rows, acc, isem, rsem):
    _sc_agg_body(False, h_hbm, src_hbm, dst_hbm, out_hbm,
                 None, sidx, didx, rows, None, acc, isem, rsem)


def _dense_body(parts, degT, h, Wl, bl, Wr, gamma, beta, out):
    deg = jnp.sum(degT[...], axis=1, keepdims=True)
    agg = (parts[0, :N, :] + parts[1, :N, :]) / jnp.maximum(deg, 1.0)
    y = (jnp.dot(agg, Wl[...], preferred_element_type=jnp.float32)
         + jnp.dot(h[...], Wr[...], preferred_element_type=jnp.float32)
         + bl[...][None, :])
    mean = jnp.mean(y, axis=0, keepdims=True)
    var = jnp.mean((y - mean) ** 2, axis=0, keepdims=True)
    yn = (y - mean) * lax.rsqrt(var + 1e-5) * gamma[...][None, :] + beta[...][None, :]
    out[...] = jnp.maximum(yn, 0.0)


def _dense(parts, degT, h, Wl, bl, Wr, gamma, beta):
    return pl.pallas_call(
        _dense_body,
        out_shape=jax.ShapeDtypeStruct((N, D), jnp.float32),
    )(parts, degT, h, Wl, bl, Wr, gamma, beta)


def kernel(x, edge_index, Wl0, bl0, Wr0, gamma0, beta0, Wl1, bl1, Wr1,
           gamma1, beta1, Wl2, bl2, Wr2, gamma2, beta2):
    src = edge_index[0]
    dst = edge_index[1]

    params = [
        (Wl0, bl0, Wr0, gamma0, beta0),
        (Wl1, bl1, Wr1, gamma1, beta1),
        (Wl2, bl2, Wr2, gamma2, beta2),
    ]

    h = x
    degT = None
    for i, (Wl, bl, Wr, gamma, beta) in enumerate(params):
        if i == 0:
            parts, deg32 = _sc_agg_deg(h, src, dst)
            degT = deg32[:, :N].T
        else:
            parts = _sc_agg(h, src, dst)
        h = _dense(parts, degT, h, Wl, bl, Wr, gamma, beta)
    return h


# speedup vs baseline: 14.2535x; 1.6023x over previous
import functools

import jax
import jax.numpy as jnp
from jax import lax
from jax.experimental import pallas as pl
from jax.experimental.pallas import tpu as pltpu
from jax.experimental.pallas import tpu_sc as plsc

N = 10000
D = 128
E = 320000
NC = 2
NS = 16
NW = NC * NS
N_PAD = 10240
ROWS_PER_TILE = N_PAD // NS
CH = 128
T0 = 78
E0 = NS * T0 * CH
C1 = (E - E0) // CH
T1_LO = C1 // NS
T1_HI = C1 - T1_LO * NS
assert E0 + C1 * CH == E and 0 <= T1_HI < NS

_mesh = plsc.VectorSubcoreMesh(core_axis_name="c", subcore_axis_name="s")
_sc_params = pltpu.CompilerParams(needs_layout_passes=False)


NROW = 2
NIDX = 4


def _sc_agg_body(compute_deg, h_hbm, ei_hbm, out_hbm,
                 deg_hbm, sidx, didx, rows, deg_v, acc, isem, rsem):
    c = lax.axis_index("c")
    s = lax.axis_index("s")
    wid = c * NS + s

    nchunks = jnp.where(c == 0, T0, T1_LO + (s < T1_HI))
    chunk0 = jnp.where(c == 0, s * T0,
                       NS * T0 + T1_LO * s + jnp.minimum(s, T1_HI))

    z16 = jnp.zeros((16,), jnp.float32)

    @pl.loop(0, CH)
    def _zrow(i):
        for k in range(D // 16):
            rows[0, i, pl.ds(k * 16, 16)] = z16

    for k in range(ROWS_PER_TILE // CH):
        pltpu.sync_copy(rows.at[0],
                        acc.at[pl.ds(s * ROWS_PER_TILE + k * CH, CH)])

    if compute_deg:
        @pl.loop(0, N_PAD // 16)
        def _zero_deg(i):
            deg_v[pl.ds(i * 16, 16)] = jnp.zeros((16,), jnp.float32)

    def issue_idx(jj, slot):
        off = (chunk0 + jj) * CH
        pltpu.async_copy(ei_hbm.at[0, pl.ds(off, CH)], sidx.at[slot],
                         isem.at[slot])
        pltpu.async_copy(ei_hbm.at[1, pl.ds(off, CH)], didx.at[slot],
                         isem.at[slot])

    def wait_idx(jj, slot):
        off = (chunk0 + jj) * CH
        pltpu.make_async_copy(ei_hbm.at[0, pl.ds(off, CH)], sidx.at[slot],
                              isem.at[slot]).wait()
        pltpu.make_async_copy(ei_hbm.at[1, pl.ds(off, CH)], didx.at[slot],
                              isem.at[slot]).wait()

    def issue_gather(slot_i, slot_r):
        pltpu.async_copy(h_hbm.at[sidx.at[slot_i]], rows.at[slot_r],
                         rsem.at[slot_r])

    def wait_gather(slot_i, slot_r):
        pltpu.make_async_copy(h_hbm.at[sidx.at[slot_i]], rows.at[slot_r],
                              rsem.at[slot_r]).wait()

    plsc.subcore_barrier()

    for j in range(NIDX):
        issue_idx(j, j)
    for j in range(NROW):
        wait_idx(j, j)
        issue_gather(j, j)

    if compute_deg:
        ones16 = jnp.ones((16,), jnp.float32)

    @pl.loop(0, nchunks, step=NIDX)
    def _group(j0):
        for b4 in range(NIDX):
            j = j0 + b4

            @pl.when(j < nchunks)
            def _():
                br = b4 % NROW
                wait_gather(b4, br)
                pltpu.sync_copy(rows.at[br], acc.at[didx.at[b4]], add=True)
                if compute_deg:
                    @pl.loop(0, CH // 16)
                    def _deg(k):
                        idx = didx[b4, pl.ds(k * 16, 16)]
                        plsc.addupdate_scatter(deg_v, [idx], ones16)

                @pl.when(j + NROW < nchunks)
                def _():
                    wait_idx(j + NROW, (b4 + NROW) % NIDX)
                    issue_gather((b4 + NROW) % NIDX, br)

                @pl.when(j + NIDX < nchunks)
                def _():
                    issue_idx(j + NIDX, b4)

    plsc.subcore_barrier()

    sl = pl.ds(s * ROWS_PER_TILE, ROWS_PER_TILE)
    pltpu.sync_copy(acc.at[sl], out_hbm.at[c, sl])
    if compute_deg:
        pltpu.sync_copy(deg_v, deg_hbm.at[wid])


@functools.partial(
    pl.kernel,
    mesh=_mesh,
    out_type=(
        jax.ShapeDtypeStruct((NC, N_PAD, D), jnp.float32),
        jax.ShapeDtypeStruct((NW, N_PAD), jnp.float32),
    ),
    scratch_types=[
        pltpu.VMEM((NIDX, CH), jnp.int32),
        pltpu.VMEM((NIDX, CH), jnp.int32),
        pltpu.VMEM((NROW, CH, D), jnp.float32),
        pltpu.VMEM((N_PAD,), jnp.float32),
        pltpu.VMEM_SHARED((N_PAD, D), jnp.float32),
        pltpu.SemaphoreType.DMA((NIDX,)),
        pltpu.SemaphoreType.DMA((NROW,)),
    ],
    compiler_params=_sc_params,
)
def _sc_agg_deg(h_hbm, ei_hbm, out_hbm, deg_hbm,
                sidx, didx, rows, deg_v, acc, isem, rsem):
    _sc_agg_body(True, h_hbm, ei_hbm, out_hbm,
                 deg_hbm, sidx, didx, rows, deg_v, acc, isem, rsem)


@functools.partial(
    pl.kernel,
    mesh=_mesh,
    out_type=jax.ShapeDtypeStruct((NC, N_PAD, D), jnp.float32),
    scratch_types=[
        pltpu.VMEM((NIDX, CH), jnp.int32),
        pltpu.VMEM((NIDX, CH), jnp.int32),
        pltpu.VMEM((NROW, CH, D), jnp.float32),
        pltpu.VMEM_SHARED((N_PAD, D), jnp.float32),
        pltpu.SemaphoreType.DMA((NIDX,)),
        pltpu.SemaphoreType.DMA((NROW,)),
    ],
    compiler_params=_sc_params,
)
def _sc_agg(h_hbm, ei_hbm, out_hbm,
            sidx, didx, rows, acc, isem, rsem):
    _sc_agg_body(False, h_hbm, ei_hbm, out_hbm,
                 None, sidx, didx, rows, None, acc, isem, rsem)


def _dense_body(parts, degT, h, Wl, bl, Wr, gamma, beta, out):
    deg = jnp.sum(degT[...], axis=1, keepdims=True)
    agg = (parts[0, :N, :] + parts[1, :N, :]) / jnp.maximum(deg, 1.0)
    y = (jnp.dot(agg, Wl[...], preferred_element_type=jnp.float32)
         + jnp.dot(h[...], Wr[...], preferred_element_type=jnp.float32)
         + bl[...][None, :])
    mean = jnp.mean(y, axis=0, keepdims=True)
    var = jnp.mean((y - mean) ** 2, axis=0, keepdims=True)
    yn = (y - mean) * lax.rsqrt(var + 1e-5) * gamma[...][None, :] + beta[...][None, :]
    out[...] = jnp.maximum(yn, 0.0)


def _dense(parts, degT, h, Wl, bl, Wr, gamma, beta):
    return pl.pallas_call(
        _dense_body,
        out_shape=jax.ShapeDtypeStruct((N, D), jnp.float32),
    )(parts, degT, h, Wl, bl, Wr, gamma, beta)


def kernel(x, edge_index, Wl0, bl0, Wr0, gamma0, beta0, Wl1, bl1, Wr1,
           gamma1, beta1, Wl2, bl2, Wr2, gamma2, beta2):

    params = [
        (Wl0, bl0, Wr0, gamma0, beta0),
        (Wl1, bl1, Wr1, gamma1, beta1),
        (Wl2, bl2, Wr2, gamma2, beta2),
    ]

    h = x
    degT = None
    for i, (Wl, bl, Wr, gamma, beta) in enumerate(params):
        if i == 0:
            parts, deg32 = _sc_agg_deg(h, edge_index)
            degT = deg32[:, :N].T
        else:
            parts = _sc_agg(h, edge_index)
        h = _dense(parts, degT, h, Wl, bl, Wr, gamma, beta)
    return h
